# baseline (speedup 1.0000x reference)
import jax
import jax.numpy as jnp
from jax.experimental import pallas as pl

_B, _N, _MO, _D = 128, 32, 48, 64
_NB = 8


def _loop_gather(x, idx):
    acc = jnp.zeros_like(x)
    for j in range(_N):
        acc = jnp.where(idx == j, x[j:j + 1, :], acc)
    return acc


def _desc_body(cats_ref, subs_ref, mask_ref, cl_ref, cr_ref,
               ca_ref, va_ref, cb_ref, vb_ref, cnt_ref):
    cats = cats_ref[...]
    subs = subs_ref[...]
    msk = mask_ref[...]
    cl = jnp.clip(cl_ref[...], 0, _N - 1)
    cr = jnp.clip(cr_ref[...], 0, _N - 1)

    ecat = jnp.where(msk != 0, cats, 3)
    pk_l = _loop_gather(ecat + 4 * subs, cl)
    ecat_l = pk_l & 3
    subs_l = pk_l >> 2

    vpm = jnp.where(ecat == 0, subs + 1, subs_l + 1)
    cpm = jnp.where(ecat == 0, 1,
                    jnp.where((ecat == 1) & (ecat_l == 0), subs + 2, 0))

    is_after = subs == 1
    i_first = jnp.where(is_after, cr, cl)
    i_second = jnp.where(is_after, cl, cr)
    vc = vpm + 8 * cpm
    pk_f = _loop_gather(vc, i_first)
    pk_s = _loop_gather(vc, i_second)

    is_comb = ecat == 2
    c_a = jnp.where(is_comb, pk_f >> 3, cpm)
    v_a = jnp.where(is_comb, pk_f & 7, vpm)
    c_b = jnp.where(is_comb, pk_s >> 3, 0)
    v_b = pk_s & 7

    ca_ref[...] = c_a
    va_ref[...] = v_a
    cb_ref[...] = c_b
    vb_ref[...] = v_b
    cnt_ref[...] = (c_a + c_b).astype(jnp.float32)


def _expand_body(ca_ref, va_ref, cb_ref, vb_ref, e1_ref, e2_ref, out_ref):
    c_a = ca_ref[...]
    v_a = va_ref[...]
    c_b = cb_ref[...]
    v_b = vb_ref[...]
    e1 = e1_ref[...]
    e2 = e2_ref[...]
    zero = jnp.zeros((1, 1, 1, 1), jnp.float32)

    ea = jnp.where(v_a == 1, e1, jnp.where(v_a == 2, e2, zero))
    eb = jnp.where(v_b == 1, e1, jnp.where(v_b == 2, e2, zero))
    p = jax.lax.broadcasted_iota(jnp.int32, (1, _MO, 1, 1), 1)
    in_a = p < c_a
    in_ab = p < (c_a + c_b)
    out_ref[...] = jnp.where(in_a, ea, jnp.where(in_ab, eb, zero))


def kernel(node_cats, node_subs, node_mask, child_left, child_right, action_embed):
    mask_i = node_mask.astype(jnp.int32)
    col_spec = pl.BlockSpec((_N, _B), lambda: (0, 0))
    c_a, v_a, c_b, v_b, cnt_t = pl.pallas_call(
        _desc_body,
        in_specs=[col_spec] * 5,
        out_specs=[col_spec] * 5,
        out_shape=[jax.ShapeDtypeStruct((_N, _B), jnp.int32)] * 4
        + [jax.ShapeDtypeStruct((_N, _B), jnp.float32)],
    )(node_cats.T, node_subs.T, mask_i.T, child_left.T, child_right.T)

    desc_spec = pl.BlockSpec((_NB, 1, 1, _B), lambda i: (i, 0, 0, 0))
    evec_spec = pl.BlockSpec((1, 1, _D, 1), lambda i: (0, 0, 0, 0))
    out = pl.pallas_call(
        _expand_body,
        grid=(_N // _NB,),
        in_specs=[desc_spec] * 4 + [evec_spec] * 2,
        out_specs=pl.BlockSpec((_NB, _MO, _D, _B), lambda i: (i, 0, 0, 0)),
        out_shape=jax.ShapeDtypeStruct((_N, _MO, _D, _B), jnp.float32),
    )(c_a.reshape(_N, 1, 1, _B), v_a.reshape(_N, 1, 1, _B),
      c_b.reshape(_N, 1, 1, _B), v_b.reshape(_N, 1, 1, _B),
      action_embed[1].reshape(1, 1, _D, 1), action_embed[2].reshape(1, 1, _D, 1))

    return jnp.transpose(out, (3, 0, 1, 2)), cnt_t.T


# speedup vs baseline: 1.1599x; 1.1599x over previous
import jax
import jax.numpy as jnp
from jax import lax
from jax.experimental import pallas as pl

_B, _N, _MO, _D = 128, 32, 48, 64
_NB = 4


def _loop_gather(x, idx, rows):
    acc = jnp.zeros((rows, _B), x.dtype)
    for j in range(_N):
        acc = jnp.where(idx == j, x[j:j + 1, :], acc)
    return acc


def _body(cats_ref, subs_ref, mask_ref, cl_ref, cr_ref,
          catsb_ref, subsb_ref, maskb_ref, clb_ref, crb_ref,
          e1_ref, e2_ref, out_ref, cnt_ref):
    cats = cats_ref[...]
    subs = subs_ref[...]
    msk = mask_ref[...]
    cl = jnp.clip(cl_ref[...], 0, _N - 1)

    ecat = jnp.where(msk != 0, cats, 3)
    pk_l = _loop_gather(ecat + 4 * subs, cl, _N)
    vpm = jnp.where(ecat == 0, subs + 1, (pk_l >> 2) + 1)
    cpm = jnp.where(ecat == 0, 1,
                    jnp.where((ecat == 1) & ((pk_l & 3) == 0), subs + 2, 0))
    vc = vpm + 8 * cpm

    cats_b = catsb_ref[0]
    subs_b = subsb_ref[0]
    mask_b = maskb_ref[0]
    cl_b = jnp.clip(clb_ref[0], 0, _N - 1)
    cr_b = jnp.clip(crb_ref[0], 0, _N - 1)
    ecat_b = jnp.where(mask_b != 0, cats_b, 3)
    pk_lb = _loop_gather(ecat + 4 * subs, cl_b, _NB)
    vpm_b = jnp.where(ecat_b == 0, subs_b + 1, (pk_lb >> 2) + 1)
    cpm_b = jnp.where(ecat_b == 0, 1,
                      jnp.where((ecat_b == 1) & ((pk_lb & 3) == 0),
                                subs_b + 2, 0))
    is_after = subs_b == 1
    i_first = jnp.where(is_after, cr_b, cl_b)
    i_second = jnp.where(is_after, cl_b, cr_b)
    pk_f = _loop_gather(vc, i_first, _NB)
    pk_s = _loop_gather(vc, i_second, _NB)
    is_comb = ecat_b == 2
    c_a = jnp.where(is_comb, pk_f >> 3, cpm_b)
    v_a = jnp.where(is_comb, pk_f & 7, vpm_b)
    c_b = jnp.where(is_comb, pk_s >> 3, 0)
    v_b = pk_s & 7

    cnt_ref[0] = (c_a + c_b).astype(jnp.float32)

    e1 = e1_ref[...]
    e2 = e2_ref[...]
    zero = jnp.zeros((1, 1, 1), jnp.float32)
    p3 = lax.broadcasted_iota(jnp.int32, (_MO, 1, 1), 0)
    cab = c_a + c_b
    for k in range(_NB):
        ca3 = lax.broadcast_in_dim(c_a[k:k + 1, :], (1, 1, _B), (1, 2))
        cab3 = lax.broadcast_in_dim(cab[k:k + 1, :], (1, 1, _B), (1, 2))
        va3 = lax.broadcast_in_dim(v_a[k:k + 1, :], (1, 1, _B), (1, 2))
        vb3 = lax.broadcast_in_dim(v_b[k:k + 1, :], (1, 1, _B), (1, 2))
        ea = jnp.where(va3 == 1, e1, jnp.where(va3 == 2, e2, zero))
        eb = jnp.where(vb3 == 1, e1, jnp.where(vb3 == 2, e2, zero))
        in_a = p3 < ca3
        in_ab = p3 < cab3
        out_ref[k] = jnp.where(in_a, ea, jnp.where(in_ab, eb, zero))


def kernel(node_cats, node_subs, node_mask, child_left, child_right, action_embed):
    mask_i = node_mask.astype(jnp.int32)
    ct, st, mt = node_cats.T, node_subs.T, mask_i.T
    lt, rt = child_left.T, child_right.T
    g = _N // _NB
    full_spec = pl.BlockSpec((_N, _B), lambda i: (0, 0))
    blk_spec = pl.BlockSpec((1, _NB, _B), lambda i: (i, 0, 0))
    evec_spec = pl.BlockSpec((1, _D, 1), lambda i: (0, 0, 0))
    out, cnt_t = pl.pallas_call(
        _body,
        grid=(g,),
        in_specs=[full_spec] * 5 + [blk_spec] * 5 + [evec_spec] * 2,
        out_specs=[pl.BlockSpec((_NB, _MO, _D, _B), lambda i: (i, 0, 0, 0)),
                   pl.BlockSpec((1, _NB, _B), lambda i: (i, 0, 0))],
        out_shape=[jax.ShapeDtypeStruct((_N, _MO, _D, _B), jnp.float32),
                   jax.ShapeDtypeStruct((g, _NB, _B), jnp.float32)],
    )(ct, st, mt, lt, rt,
      ct.reshape(g, _NB, _B), st.reshape(g, _NB, _B), mt.reshape(g, _NB, _B),
      lt.reshape(g, _NB, _B), rt.reshape(g, _NB, _B),
      action_embed[1].reshape(1, _D, 1), action_embed[2].reshape(1, _D, 1))

    return jnp.transpose(out, (3, 0, 1, 2)), cnt_t.reshape(_N, _B).T
